# Initial kernel scaffold; baseline (speedup 1.0000x reference)
import jax
import jax.numpy as jnp
from jax.experimental import pallas as pl


def kernel(train_inputs, train_types, node_neigh, student_embeddings, course_type_embeddings, student_type_embeddings, trans_weights_s):
    raise NotImplementedError("write your pallas kernel here")


# speedup vs baseline: 15.1827x; 15.1827x over previous
import jax
import jax.numpy as jnp
import numpy as np
from jax import lax
from jax.experimental import pallas as pl
from jax.experimental.pallas import tpu as pltpu
from jax.experimental.pallas import tpu_sc as plsc

NUM_COURSES = 10000
NUM_STUDENTS = 100000
EMBED = 256
EMBED_U = 64
ETYPES = 4
NSAMP = 20
BATCH = 16384
NEIGH = ETYPES * NSAMP
TABLE_ROWS = NUM_COURSES * ETYPES + NUM_STUDENTS
STUD_OFF = NUM_COURSES * ETYPES - NUM_COURSES

NCORES = 2
NSUB = 16
NW = NCORES * NSUB
EPW = BATCH // NW
C = 8
NCHUNK = EPW // C
SLOTS = C * NEIGH
NVEC = SLOTS // 16
ROWS_PER_STREAM = 128
NSTREAM = SLOTS // ROWS_PER_STREAM


def _sc_body(neigh_hbm, train_hbm, table_hbm, stud_hbm, sums_out, gath_out,
             nb0, nb1, ib0, ib1, rb0, rb1, sb0, sb1, sr0, sr1, ss0, ss1,
             sem0, sem1):
    wid = lax.axis_index("s") * NCORES + lax.axis_index("c")
    nbs = (nb0, nb1)
    ibs = (ib0, ib1)
    rbs = (rb0, rb1)
    sbs = (sb0, sb1)
    srs = (sr0, sr1)
    sss = (ss0, ss1)
    sems = (sem0, sem1)

    def fire(k, b):
        elem_base = wid * EPW + k * C
        pltpu.sync_copy(neigh_hbm.at[pl.ds(elem_base * NEIGH, SLOTS)], nbs[b])
        pltpu.sync_copy(train_hbm.at[pl.ds(elem_base, C)], sbs[b])

        @pl.loop(0, NVEC, step=5)
        def _idx(j):
            lane = lax.iota(jnp.int32, 16)
            for r in range(5):
                n = nbs[b][pl.ds((j + r) * 16, 16)]
                off = (r * 16) % NEIGH
                base = off // NSAMP
                rem = off % NSAMP
                if rem + 16 > NSAMP:
                    e = jnp.where(lane >= (NSAMP - rem), base + 1, base)
                    course_idx = n * ETYPES + e
                else:
                    course_idx = n * ETYPES + base
                idx = jnp.where(n < NUM_COURSES, course_idx, n + STUD_OFF)
                ibs[b][pl.ds((j + r) * 16, 16)] = idx

        for s in range(NSTREAM):
            pltpu.async_copy(
                table_hbm.at[ibs[b].at[pl.ds(s * ROWS_PER_STREAM, ROWS_PER_STREAM)]],
                rbs[b].at[pl.ds(s * ROWS_PER_STREAM, ROWS_PER_STREAM)],
                sems[b])
        pltpu.async_copy(stud_hbm.at[sbs[b]], srs[b], sems[b])

    def drain(b):
        for s in range(NSTREAM):
            pltpu.make_async_copy(
                table_hbm.at[ibs[b].at[pl.ds(s * ROWS_PER_STREAM, ROWS_PER_STREAM)]],
                rbs[b].at[pl.ds(s * ROWS_PER_STREAM, ROWS_PER_STREAM)],
                sems[b]).wait()
        pltpu.make_async_copy(stud_hbm.at[sbs[b]], srs[b], sems[b]).wait()

    def flush(k, b):
        elem_base = wid * EPW + k * C
        zero = jnp.zeros((16,), jnp.float32)
        for i in range(C):
            def red(r, acc):
                row = i * NEIGH + r
                return tuple(acc[d] + rbs[b][row, pl.ds(d * 16, 16)]
                             for d in range(4))
            acc = pl.loop(0, NEIGH, init_carry=(zero,) * 4, unroll=4)(red)
            for d in range(4):
                sss[b][i, pl.ds(d * 16, 16)] = acc[d]
        pltpu.sync_copy(sss[b], sums_out.at[pl.ds(elem_base, C)])
        pltpu.sync_copy(srs[b], gath_out.at[pl.ds(elem_base, C)])

    fire(0, 0)

    @pl.loop(0, NCHUNK, step=2)
    def _outer(k2):
        for b in range(2):
            k = k2 + b

            @pl.when(k + 1 < NCHUNK)
            def _():
                fire(k + 1, 1 - b)

            drain(b)
            flush(k, b)


def _sc_gather(neigh, train_idx, table, stud_table):
    mesh = plsc.VectorSubcoreMesh(core_axis_name="c", subcore_axis_name="s")
    f = pl.kernel(
        _sc_body,
        out_type=[
            jax.ShapeDtypeStruct((BATCH, EMBED_U), jnp.float32),
            jax.ShapeDtypeStruct((BATCH, EMBED), jnp.float32),
        ],
        mesh=mesh,
        scratch_types=[
            pltpu.VMEM((SLOTS,), jnp.int32),
            pltpu.VMEM((SLOTS,), jnp.int32),
            pltpu.VMEM((SLOTS,), jnp.int32),
            pltpu.VMEM((SLOTS,), jnp.int32),
            pltpu.VMEM((SLOTS, EMBED_U), jnp.float32),
            pltpu.VMEM((SLOTS, EMBED_U), jnp.float32),
            pltpu.VMEM((C,), jnp.int32),
            pltpu.VMEM((C,), jnp.int32),
            pltpu.VMEM((C, EMBED), jnp.float32),
            pltpu.VMEM((C, EMBED), jnp.float32),
            pltpu.VMEM((C, EMBED_U), jnp.float32),
            pltpu.VMEM((C, EMBED_U), jnp.float32),
            pltpu.SemaphoreType.DMA,
            pltpu.SemaphoreType.DMA,
        ],
        compiler_params=pltpu.CompilerParams(use_tc_tiling_on_sc=False),
    )
    return f(neigh, train_idx, table, stud_table)


def _tc_body(sums_ref, stud_ref, w_ref, out_ref):
    s = sums_ref[...] * (1.0 / NEIGH)
    y = stud_ref[...] + jnp.dot(s, w_ref[...],
                                preferred_element_type=jnp.float32)
    nrm = jnp.sqrt(jnp.sum(y * y, axis=1, keepdims=True))
    out_ref[...] = y / jnp.maximum(nrm, 1e-12)


def _tc_finish(sums, gath, w):
    blk = 1024
    return pl.pallas_call(
        _tc_body,
        grid=(BATCH // blk,),
        in_specs=[
            pl.BlockSpec((blk, EMBED_U), lambda i: (i, 0)),
            pl.BlockSpec((blk, EMBED), lambda i: (i, 0)),
            pl.BlockSpec((EMBED_U, EMBED), lambda i: (0, 0)),
        ],
        out_specs=pl.BlockSpec((blk, EMBED), lambda i: (i, 0)),
        out_shape=jax.ShapeDtypeStruct((BATCH, EMBED), jnp.float32),
    )(sums, gath, w)


def kernel(train_inputs, train_types, node_neigh, student_embeddings,
           course_type_embeddings, student_type_embeddings, trans_weights_s):
    del train_types
    neigh = node_neigh.astype(jnp.int32).reshape(-1)
    tin = train_inputs.astype(jnp.int32)
    table = jnp.concatenate(
        [course_type_embeddings.reshape(NUM_COURSES * ETYPES, EMBED_U),
         student_type_embeddings.reshape(NUM_STUDENTS, EMBED_U)], axis=0)
    sums, gath = _sc_gather(neigh, tin, table, student_embeddings)
    return _tc_finish(sums, gath, trans_weights_s[0])


# speedup vs baseline: 18.4835x; 1.2174x over previous
import jax
import jax.numpy as jnp
import numpy as np
from jax import lax
from jax.experimental import pallas as pl
from jax.experimental.pallas import tpu as pltpu
from jax.experimental.pallas import tpu_sc as plsc

NUM_COURSES = 10000
NUM_STUDENTS = 100000
EMBED = 256
EMBED_U = 64
ETYPES = 4
NSAMP = 20
BATCH = 16384
NEIGH = ETYPES * NSAMP
TABLE_ROWS = NUM_COURSES * ETYPES + NUM_STUDENTS
STUD_OFF = NUM_COURSES * ETYPES - NUM_COURSES

NCORES = 2
NSUB = 16
NW = NCORES * NSUB
EPW = BATCH // NW
C = 8
NCHUNK = EPW // C
SLOTS = C * NEIGH
NVEC = SLOTS // 16
ROWS_PER_STREAM = 128
NSTREAM = SLOTS // ROWS_PER_STREAM


def _sc_body(neigh_hbm, table_hbm, sums_out,
             nb0, nb1, ib0, ib1, rb0, rb1, ss0, ss1,
             sem0, sem1):
    wid = lax.axis_index("s") * NCORES + lax.axis_index("c")
    nbs = (nb0, nb1)
    ibs = (ib0, ib1)
    rbs = (rb0, rb1)
    sss = (ss0, ss1)
    sems = (sem0, sem1)

    def fire(k, b):
        elem_base = wid * EPW + k * C
        pltpu.sync_copy(neigh_hbm.at[pl.ds(elem_base * NEIGH, SLOTS)], nbs[b])

        @pl.loop(0, NVEC, step=5)
        def _idx(j):
            lane = lax.iota(jnp.int32, 16)
            for r in range(5):
                n = nbs[b][pl.ds((j + r) * 16, 16)]
                off = (r * 16) % NEIGH
                base = off // NSAMP
                rem = off % NSAMP
                if rem + 16 > NSAMP:
                    e = jnp.where(lane >= (NSAMP - rem), base + 1, base)
                    course_idx = n * ETYPES + e
                else:
                    course_idx = n * ETYPES + base
                idx = jnp.where(n < NUM_COURSES, course_idx, n + STUD_OFF)
                ibs[b][pl.ds((j + r) * 16, 16)] = idx

        for s in range(NSTREAM):
            pltpu.async_copy(
                table_hbm.at[ibs[b].at[pl.ds(s * ROWS_PER_STREAM, ROWS_PER_STREAM)]],
                rbs[b].at[pl.ds(s * ROWS_PER_STREAM, ROWS_PER_STREAM)],
                sems[b])

    def drain(b):
        for s in range(NSTREAM):
            pltpu.make_async_copy(
                table_hbm.at[ibs[b].at[pl.ds(s * ROWS_PER_STREAM, ROWS_PER_STREAM)]],
                rbs[b].at[pl.ds(s * ROWS_PER_STREAM, ROWS_PER_STREAM)],
                sems[b]).wait()

    def flush(k, b):
        elem_base = wid * EPW + k * C
        zero = jnp.zeros((16,), jnp.float32)
        for i in range(C):
            def red(r, acc):
                row = i * NEIGH + r
                return tuple(acc[d] + rbs[b][row, pl.ds(d * 16, 16)]
                             for d in range(4))
            acc = pl.loop(0, NEIGH, init_carry=(zero,) * 4, unroll=4)(red)
            for d in range(4):
                sss[b][i, pl.ds(d * 16, 16)] = acc[d]
        pltpu.sync_copy(sss[b], sums_out.at[pl.ds(elem_base, C)])

    fire(0, 0)

    @pl.loop(0, NCHUNK, step=2)
    def _outer(k2):
        for b in range(2):
            k = k2 + b

            @pl.when(k + 1 < NCHUNK)
            def _():
                fire(k + 1, 1 - b)

            drain(b)
            flush(k, b)


SCHUNK = 128
NSCHUNK = EPW // SCHUNK


def _sc_stud_body(train_hbm, stud_hbm, gath_out, sb0, sb1, sr0, sr1,
                  sem0, sem1):
    wid = lax.axis_index("s") * NCORES + lax.axis_index("c")
    sbs = (sb0, sb1)
    srs = (sr0, sr1)
    sems = (sem0, sem1)

    def fire(k, b):
        base = wid * EPW + k * SCHUNK
        pltpu.sync_copy(train_hbm.at[pl.ds(base, SCHUNK)], sbs[b])
        pltpu.async_copy(stud_hbm.at[sbs[b]], srs[b], sems[b])

    fire(0, 0)

    @pl.loop(0, NSCHUNK, step=2)
    def _outer(k2):
        for b in range(2):
            k = k2 + b

            @pl.when(k + 1 < NSCHUNK)
            def _():
                fire(k + 1, 1 - b)

            pltpu.make_async_copy(stud_hbm.at[sbs[b]], srs[b],
                                  sems[b]).wait()
            base = wid * EPW + k * SCHUNK
            pltpu.sync_copy(srs[b], gath_out.at[pl.ds(base, SCHUNK)])


def _sc_gather(neigh, train_idx, table, stud_table):
    mesh = plsc.VectorSubcoreMesh(core_axis_name="c", subcore_axis_name="s")
    f = pl.kernel(
        _sc_body,
        out_type=jax.ShapeDtypeStruct((BATCH, EMBED_U), jnp.float32),
        mesh=mesh,
        scratch_types=[
            pltpu.VMEM((SLOTS,), jnp.int32),
            pltpu.VMEM((SLOTS,), jnp.int32),
            pltpu.VMEM((SLOTS,), jnp.int32),
            pltpu.VMEM((SLOTS,), jnp.int32),
            pltpu.VMEM((SLOTS, EMBED_U), jnp.float32),
            pltpu.VMEM((SLOTS, EMBED_U), jnp.float32),
            pltpu.VMEM((C, EMBED_U), jnp.float32),
            pltpu.VMEM((C, EMBED_U), jnp.float32),
            pltpu.SemaphoreType.DMA,
            pltpu.SemaphoreType.DMA,
        ],
        compiler_params=pltpu.CompilerParams(use_tc_tiling_on_sc=False),
    )
    sums = f(neigh, table)

    g = pl.kernel(
        _sc_stud_body,
        out_type=jax.ShapeDtypeStruct((BATCH, EMBED), jnp.float32),
        mesh=plsc.VectorSubcoreMesh(core_axis_name="c", subcore_axis_name="s"),
        scratch_types=[
            pltpu.VMEM((SCHUNK,), jnp.int32),
            pltpu.VMEM((SCHUNK,), jnp.int32),
            pltpu.VMEM((SCHUNK, EMBED), jnp.float32),
            pltpu.VMEM((SCHUNK, EMBED), jnp.float32),
            pltpu.SemaphoreType.DMA,
            pltpu.SemaphoreType.DMA,
        ],
    )
    gath = g(train_idx, stud_table)
    return sums, gath


def _tc_body(sums_ref, stud_ref, w_ref, out_ref):
    s = sums_ref[...] * (1.0 / NEIGH)
    y = stud_ref[...] + jnp.dot(s, w_ref[...],
                                preferred_element_type=jnp.float32)
    nrm = jnp.sqrt(jnp.sum(y * y, axis=1, keepdims=True))
    out_ref[...] = y / jnp.maximum(nrm, 1e-12)


def _tc_finish(sums, gath, w):
    blk = 1024
    return pl.pallas_call(
        _tc_body,
        grid=(BATCH // blk,),
        in_specs=[
            pl.BlockSpec((blk, EMBED_U), lambda i: (i, 0)),
            pl.BlockSpec((blk, EMBED), lambda i: (i, 0)),
            pl.BlockSpec((EMBED_U, EMBED), lambda i: (0, 0)),
        ],
        out_specs=pl.BlockSpec((blk, EMBED), lambda i: (i, 0)),
        out_shape=jax.ShapeDtypeStruct((BATCH, EMBED), jnp.float32),
    )(sums, gath, w)


def kernel(train_inputs, train_types, node_neigh, student_embeddings,
           course_type_embeddings, student_type_embeddings, trans_weights_s):
    del train_types
    neigh = node_neigh.astype(jnp.int32).reshape(-1)
    tin = train_inputs.astype(jnp.int32)
    table = jnp.concatenate(
        [course_type_embeddings.reshape(NUM_COURSES * ETYPES, EMBED_U),
         student_type_embeddings.reshape(NUM_STUDENTS, EMBED_U)], axis=0)
    sums, gath = _sc_gather(neigh, tin, table, student_embeddings)
    return _tc_finish(sums, gath, trans_weights_s[0])


# speedup vs baseline: 19.7177x; 1.0668x over previous
import jax
import jax.numpy as jnp
import numpy as np
from jax import lax
from jax.experimental import pallas as pl
from jax.experimental.pallas import tpu as pltpu
from jax.experimental.pallas import tpu_sc as plsc

NUM_COURSES = 10000
NUM_STUDENTS = 100000
EMBED = 256
EMBED_U = 64
ETYPES = 4
NSAMP = 20
BATCH = 16384
NEIGH = ETYPES * NSAMP
TABLE_ROWS = NUM_COURSES * ETYPES + NUM_STUDENTS
STUD_OFF = NUM_COURSES * ETYPES - NUM_COURSES

NCORES = 2
NSUB = 16
NW = NCORES * NSUB
EPW = BATCH // NW
C = 8
NCHUNK = EPW // C
SLOTS = C * NEIGH
NVEC = SLOTS // 16
ROWS_PER_STREAM = 128
NSTREAM = SLOTS // ROWS_PER_STREAM


def _sc_body(neigh_hbm, table_hbm, sums_out,
             nb0, nb1, ib0, ib1, rb0, rb1, ss0, ss1,
             nsem0, nsem1, gsem0, gsem1, osem0, osem1):
    wid = lax.axis_index("s") * NCORES + lax.axis_index("c")
    nbs = (nb0, nb1)
    ibs = (ib0, ib1)
    rbs = (rb0, rb1)
    sss = (ss0, ss1)
    nsems = (nsem0, nsem1)
    gsems = (gsem0, gsem1)
    osems = (osem0, osem1)

    def neigh_slice(k):
        return neigh_hbm.at[pl.ds((wid * EPW + k * C) * NEIGH, SLOTS)]

    def fire_neigh(k, b):
        pltpu.async_copy(neigh_slice(k), nbs[b], nsems[b])

    def fire_gathers(k, b):
        pltpu.make_async_copy(neigh_slice(k), nbs[b], nsems[b]).wait()

        @pl.loop(0, NVEC, step=5)
        def _idx(j):
            lane = lax.iota(jnp.int32, 16)
            for r in range(5):
                n = nbs[b][pl.ds((j + r) * 16, 16)]
                off = (r * 16) % NEIGH
                base = off // NSAMP
                rem = off % NSAMP
                if rem + 16 > NSAMP:
                    e = jnp.where(lane >= (NSAMP - rem), base + 1, base)
                    course_idx = n * ETYPES + e
                else:
                    course_idx = n * ETYPES + base
                idx = jnp.where(n < NUM_COURSES, course_idx, n + STUD_OFF)
                ibs[b][pl.ds((j + r) * 16, 16)] = idx

        for s in range(NSTREAM):
            pltpu.async_copy(
                table_hbm.at[ibs[b].at[pl.ds(s * ROWS_PER_STREAM, ROWS_PER_STREAM)]],
                rbs[b].at[pl.ds(s * ROWS_PER_STREAM, ROWS_PER_STREAM)],
                gsems[b])

    def drain_gathers(b):
        for s in range(NSTREAM):
            pltpu.make_async_copy(
                table_hbm.at[ibs[b].at[pl.ds(s * ROWS_PER_STREAM, ROWS_PER_STREAM)]],
                rbs[b].at[pl.ds(s * ROWS_PER_STREAM, ROWS_PER_STREAM)],
                gsems[b]).wait()

    def sums_slice(k):
        return sums_out.at[pl.ds(wid * EPW + k * C, C)]

    def flush(k, b):
        drain_gathers(b)
        zero = jnp.zeros((16,), jnp.float32)
        accs = []
        for i in range(C):
            def red(r, acc):
                row = i * NEIGH + r
                return tuple(acc[d] + rbs[b][row, pl.ds(d * 16, 16)]
                             for d in range(4))
            accs.append(pl.loop(0, NEIGH, init_carry=(zero,) * 4,
                                unroll=4)(red))

        @pl.when(k >= 2)
        def _():
            pltpu.make_async_copy(sss[b], sums_slice(k - 2), osems[b]).wait()

        for i in range(C):
            for d in range(4):
                sss[b][i, pl.ds(d * 16, 16)] = accs[i][d]
        pltpu.async_copy(sss[b], sums_slice(k), osems[b])

    fire_neigh(0, 0)
    fire_neigh(1, 1)
    fire_gathers(0, 0)

    @pl.loop(0, NCHUNK, step=2)
    def _outer(k2):
        for b in range(2):
            k = k2 + b

            @pl.when(k + 2 < NCHUNK)
            def _():
                fire_neigh(k + 2, b)

            @pl.when(k + 1 < NCHUNK)
            def _():
                fire_gathers(k + 1, 1 - b)

            flush(k, b)

    for b in range(2):
        pltpu.make_async_copy(sss[b], sums_slice(NCHUNK - 2 + b),
                              osems[b]).wait()


SCHUNK = 128
NSCHUNK = EPW // SCHUNK


def _sc_stud_body(train_hbm, stud_hbm, gath_out, sb0, sb1, sr0, sr1,
                  sem0, sem1):
    wid = lax.axis_index("s") * NCORES + lax.axis_index("c")
    sbs = (sb0, sb1)
    srs = (sr0, sr1)
    sems = (sem0, sem1)

    def fire(k, b):
        base = wid * EPW + k * SCHUNK
        pltpu.sync_copy(train_hbm.at[pl.ds(base, SCHUNK)], sbs[b])
        pltpu.async_copy(stud_hbm.at[sbs[b]], srs[b], sems[b])

    fire(0, 0)

    @pl.loop(0, NSCHUNK, step=2)
    def _outer(k2):
        for b in range(2):
            k = k2 + b

            @pl.when(k + 1 < NSCHUNK)
            def _():
                fire(k + 1, 1 - b)

            pltpu.make_async_copy(stud_hbm.at[sbs[b]], srs[b],
                                  sems[b]).wait()
            base = wid * EPW + k * SCHUNK
            pltpu.sync_copy(srs[b], gath_out.at[pl.ds(base, SCHUNK)])


def _sc_gather(neigh, train_idx, table, stud_table):
    mesh = plsc.VectorSubcoreMesh(core_axis_name="c", subcore_axis_name="s")
    f = pl.kernel(
        _sc_body,
        out_type=jax.ShapeDtypeStruct((BATCH, EMBED_U), jnp.float32),
        mesh=mesh,
        scratch_types=[
            pltpu.VMEM((SLOTS,), jnp.int32),
            pltpu.VMEM((SLOTS,), jnp.int32),
            pltpu.VMEM((SLOTS,), jnp.int32),
            pltpu.VMEM((SLOTS,), jnp.int32),
            pltpu.VMEM((SLOTS, EMBED_U), jnp.float32),
            pltpu.VMEM((SLOTS, EMBED_U), jnp.float32),
            pltpu.VMEM((C, EMBED_U), jnp.float32),
            pltpu.VMEM((C, EMBED_U), jnp.float32),
            pltpu.SemaphoreType.DMA,
            pltpu.SemaphoreType.DMA,
            pltpu.SemaphoreType.DMA,
            pltpu.SemaphoreType.DMA,
            pltpu.SemaphoreType.DMA,
            pltpu.SemaphoreType.DMA,
        ],
        compiler_params=pltpu.CompilerParams(use_tc_tiling_on_sc=False),
    )
    sums = f(neigh, table)

    g = pl.kernel(
        _sc_stud_body,
        out_type=jax.ShapeDtypeStruct((BATCH, EMBED), jnp.float32),
        mesh=plsc.VectorSubcoreMesh(core_axis_name="c", subcore_axis_name="s"),
        scratch_types=[
            pltpu.VMEM((SCHUNK,), jnp.int32),
            pltpu.VMEM((SCHUNK,), jnp.int32),
            pltpu.VMEM((SCHUNK, EMBED), jnp.float32),
            pltpu.VMEM((SCHUNK, EMBED), jnp.float32),
            pltpu.SemaphoreType.DMA,
            pltpu.SemaphoreType.DMA,
        ],
    )
    gath = g(train_idx, stud_table)
    return sums, gath


def _tc_body(sums_ref, stud_ref, w_ref, out_ref):
    s = sums_ref[...] * (1.0 / NEIGH)
    y = stud_ref[...] + jnp.dot(s, w_ref[...],
                                preferred_element_type=jnp.float32)
    nrm = jnp.sqrt(jnp.sum(y * y, axis=1, keepdims=True))
    out_ref[...] = y / jnp.maximum(nrm, 1e-12)


def _tc_finish(sums, gath, w):
    blk = 1024
    return pl.pallas_call(
        _tc_body,
        grid=(BATCH // blk,),
        in_specs=[
            pl.BlockSpec((blk, EMBED_U), lambda i: (i, 0)),
            pl.BlockSpec((blk, EMBED), lambda i: (i, 0)),
            pl.BlockSpec((EMBED_U, EMBED), lambda i: (0, 0)),
        ],
        out_specs=pl.BlockSpec((blk, EMBED), lambda i: (i, 0)),
        out_shape=jax.ShapeDtypeStruct((BATCH, EMBED), jnp.float32),
    )(sums, gath, w)


def kernel(train_inputs, train_types, node_neigh, student_embeddings,
           course_type_embeddings, student_type_embeddings, trans_weights_s):
    del train_types
    neigh = node_neigh.astype(jnp.int32).reshape(-1)
    tin = train_inputs.astype(jnp.int32)
    table = jnp.concatenate(
        [course_type_embeddings.reshape(NUM_COURSES * ETYPES, EMBED_U),
         student_type_embeddings.reshape(NUM_STUDENTS, EMBED_U)], axis=0)
    sums, gath = _sc_gather(neigh, tin, table, student_embeddings)
    return _tc_finish(sums, gath, trans_weights_s[0])


# speedup vs baseline: 20.6265x; 1.0461x over previous
import jax
import jax.numpy as jnp
import numpy as np
from jax import lax
from jax.experimental import pallas as pl
from jax.experimental.pallas import tpu as pltpu
from jax.experimental.pallas import tpu_sc as plsc

NUM_COURSES = 10000
NUM_STUDENTS = 100000
EMBED = 256
EMBED_U = 64
ETYPES = 4
NSAMP = 20
BATCH = 16384
NEIGH = ETYPES * NSAMP
TABLE_ROWS = NUM_COURSES * ETYPES + NUM_STUDENTS
STUD_OFF = NUM_COURSES * ETYPES - NUM_COURSES

NCORES = 2
NSUB = 16
NW = NCORES * NSUB
EPW = BATCH // NW
C = 16
NCHUNK = EPW // C
SLOTS = C * NEIGH
HALF = SLOTS // 2
ROWS_PER_STREAM = 128
NSTREAM = HALF // ROWS_PER_STREAM


def _sc_body(neigh_hbm, table_hbm, sums_out,
             nb0, nb1, ib0, ib1, rb0, rb1, ss0, ss1,
             nsem0, nsem1, gsem0, gsem1, osem0, osem1):
    wid = lax.axis_index("s") * NCORES + lax.axis_index("c")
    nbs = (nb0, nb1)
    ibs = (ib0, ib1)
    rbs = (rb0, rb1)
    sss = (ss0, ss1)
    nsems = (nsem0, nsem1)
    gsems = (gsem0, gsem1)
    osems = (osem0, osem1)

    def neigh_slice(c):
        return neigh_hbm.at[:, :, pl.ds(wid * EPW + c * C, C)]

    def fire_neigh(c, p):
        pltpu.async_copy(neigh_slice(c), nbs[p], nsems[p])

    def compute_idx(c, p):
        pltpu.make_async_copy(neigh_slice(c), nbs[p], nsems[p]).wait()
        nb = nbs[p]
        ib = ibs[p]

        @pl.loop(0, NSAMP)
        def _s(sj):
            for e in range(ETYPES):
                n = nb[sj, e, pl.ds(0, C)]
                idx = jnp.where(n < NUM_COURSES, n * ETYPES + e, n + STUD_OFF)
                ib[pl.ds(sj * (ETYPES * C) + e * C, C)] = idx

    def fire_half(p, hb):
        ib = ibs[p]
        for t in range(NSTREAM):
            pltpu.async_copy(
                table_hbm.at[ib.at[pl.ds(hb * HALF + t * ROWS_PER_STREAM,
                                         ROWS_PER_STREAM)]],
                rbs[hb].at[pl.ds(t * ROWS_PER_STREAM, ROWS_PER_STREAM)],
                gsems[hb])

    def drain_half(p, hb):
        ib = ibs[p]
        for t in range(NSTREAM):
            pltpu.make_async_copy(
                table_hbm.at[ib.at[pl.ds(hb * HALF + t * ROWS_PER_STREAM,
                                         ROWS_PER_STREAM)]],
                rbs[hb].at[pl.ds(t * ROWS_PER_STREAM, ROWS_PER_STREAM)],
                gsems[hb]).wait()

    def sums_slice(c):
        return sums_out.at[pl.ds(wid * EPW + c * C, C)]

    def acc_pass(p, hb):
        rb = rbs[hb]
        ss = sss[p]
        zero = jnp.zeros((16,), jnp.float32)
        for i in range(C):
            def red(r, acc):
                row = r * C + i
                return tuple(acc[d] + rb[row, pl.ds(d * 16, 16)]
                             for d in range(4))
            acc = pl.loop(0, NEIGH // 2, init_carry=(zero,) * 4,
                          unroll=4)(red)
            for d in range(4):
                if hb == 0:
                    ss[i, pl.ds(d * 16, 16)] = acc[d]
                else:
                    ss[i, pl.ds(d * 16, 16)] = (
                        ss[i, pl.ds(d * 16, 16)] + acc[d])

    fire_neigh(0, 0)
    fire_neigh(1, 1)
    compute_idx(0, 0)
    fire_half(0, 0)

    @pl.loop(0, NCHUNK, step=2)
    def _outer(c2):
        for cb in range(2):
            c = c2 + cb

            @pl.when(c + 2 < NCHUNK)
            def _():
                fire_neigh(c + 2, cb)

            @pl.when(c >= 2)
            def _():
                pltpu.make_async_copy(sss[cb], sums_slice(c - 2),
                                      osems[cb]).wait()

            drain_half(cb, 0)
            fire_half(cb, 1)
            acc_pass(cb, 0)

            @pl.when(c + 1 < NCHUNK)
            def _():
                compute_idx(c + 1, 1 - cb)

            drain_half(cb, 1)

            @pl.when(c + 1 < NCHUNK)
            def _():
                fire_half(1 - cb, 0)

            acc_pass(cb, 1)
            pltpu.async_copy(sss[cb], sums_slice(c), osems[cb])

    for b in range(2):
        pltpu.make_async_copy(sss[b], sums_slice(NCHUNK - 2 + b),
                              osems[b]).wait()


SCHUNK = 128
NSCHUNK = EPW // SCHUNK


def _sc_stud_body(train_hbm, stud_hbm, gath_out, sb0, sb1, sr0, sr1,
                  sem0, sem1):
    wid = lax.axis_index("s") * NCORES + lax.axis_index("c")
    sbs = (sb0, sb1)
    srs = (sr0, sr1)
    sems = (sem0, sem1)

    def fire(k, b):
        base = wid * EPW + k * SCHUNK
        pltpu.sync_copy(train_hbm.at[pl.ds(base, SCHUNK)], sbs[b])
        pltpu.async_copy(stud_hbm.at[sbs[b]], srs[b], sems[b])

    fire(0, 0)

    @pl.loop(0, NSCHUNK, step=2)
    def _outer(k2):
        for b in range(2):
            k = k2 + b

            @pl.when(k + 1 < NSCHUNK)
            def _():
                fire(k + 1, 1 - b)

            pltpu.make_async_copy(stud_hbm.at[sbs[b]], srs[b],
                                  sems[b]).wait()
            base = wid * EPW + k * SCHUNK
            pltpu.sync_copy(srs[b], gath_out.at[pl.ds(base, SCHUNK)])


def _sc_gather(neigh, train_idx, table, stud_table):
    mesh = plsc.VectorSubcoreMesh(core_axis_name="c", subcore_axis_name="s")
    f = pl.kernel(
        _sc_body,
        out_type=jax.ShapeDtypeStruct((BATCH, EMBED_U), jnp.float32),
        mesh=mesh,
        scratch_types=[
            pltpu.VMEM((NSAMP, ETYPES, C), jnp.int32),
            pltpu.VMEM((NSAMP, ETYPES, C), jnp.int32),
            pltpu.VMEM((SLOTS,), jnp.int32),
            pltpu.VMEM((SLOTS,), jnp.int32),
            pltpu.VMEM((HALF, EMBED_U), jnp.float32),
            pltpu.VMEM((HALF, EMBED_U), jnp.float32),
            pltpu.VMEM((C, EMBED_U), jnp.float32),
            pltpu.VMEM((C, EMBED_U), jnp.float32),
            pltpu.SemaphoreType.DMA,
            pltpu.SemaphoreType.DMA,
            pltpu.SemaphoreType.DMA,
            pltpu.SemaphoreType.DMA,
            pltpu.SemaphoreType.DMA,
            pltpu.SemaphoreType.DMA,
        ],
        compiler_params=pltpu.CompilerParams(use_tc_tiling_on_sc=False),
    )
    sums = f(neigh, table)

    g = pl.kernel(
        _sc_stud_body,
        out_type=jax.ShapeDtypeStruct((BATCH, EMBED), jnp.float32),
        mesh=plsc.VectorSubcoreMesh(core_axis_name="c", subcore_axis_name="s"),
        scratch_types=[
            pltpu.VMEM((SCHUNK,), jnp.int32),
            pltpu.VMEM((SCHUNK,), jnp.int32),
            pltpu.VMEM((SCHUNK, EMBED), jnp.float32),
            pltpu.VMEM((SCHUNK, EMBED), jnp.float32),
            pltpu.SemaphoreType.DMA,
            pltpu.SemaphoreType.DMA,
        ],
    )
    gath = g(train_idx, stud_table)
    return sums, gath


def _tc_body(sums_ref, stud_ref, w_ref, out_ref):
    s = sums_ref[...] * (1.0 / NEIGH)
    y = stud_ref[...] + jnp.dot(s, w_ref[...],
                                preferred_element_type=jnp.float32)
    nrm = jnp.sqrt(jnp.sum(y * y, axis=1, keepdims=True))
    out_ref[...] = y / jnp.maximum(nrm, 1e-12)


def _tc_finish(sums, gath, w):
    blk = 1024
    return pl.pallas_call(
        _tc_body,
        grid=(BATCH // blk,),
        in_specs=[
            pl.BlockSpec((blk, EMBED_U), lambda i: (i, 0)),
            pl.BlockSpec((blk, EMBED), lambda i: (i, 0)),
            pl.BlockSpec((EMBED_U, EMBED), lambda i: (0, 0)),
        ],
        out_specs=pl.BlockSpec((blk, EMBED), lambda i: (i, 0)),
        out_shape=jax.ShapeDtypeStruct((BATCH, EMBED), jnp.float32),
    )(sums, gath, w)


def kernel(train_inputs, train_types, node_neigh, student_embeddings,
           course_type_embeddings, student_type_embeddings, trans_weights_s):
    del train_types
    neigh = jnp.transpose(node_neigh.astype(jnp.int32), (2, 1, 0))
    tin = train_inputs.astype(jnp.int32)
    table = jnp.concatenate(
        [course_type_embeddings.reshape(NUM_COURSES * ETYPES, EMBED_U),
         student_type_embeddings.reshape(NUM_STUDENTS, EMBED_U)], axis=0)
    sums, gath = _sc_gather(neigh, tin, table, student_embeddings)
    return _tc_finish(sums, gath, trans_weights_s[0])
